# baseline (speedup 1.0000x reference)
import functools

import jax
import jax.numpy as jnp
from jax import lax
from jax.experimental import pallas as pl
from jax.experimental.pallas import tpu as pltpu
from jax.experimental.pallas import tpu_sc as plsc

B = 16384
EMB = 64
L = 16
NC = 2
NS = 16
NW = NC * NS
BPW = B // NW
CH = 256
NCH = BPW // CH

_mesh = plsc.VectorSubcoreMesh(core_axis_name="c", subcore_axis_name="s")


@functools.partial(
    pl.kernel,
    mesh=_mesh,
    compiler_params=pltpu.CompilerParams(
        needs_layout_passes=False, use_tc_tiling_on_sc=True),
    out_type=jax.ShapeDtypeStruct((B,), jnp.float32),
    scratch_types=[
        pltpu.VMEM((BPW,), jnp.int32),
        pltpu.VMEM((BPW,), jnp.int32),
        pltpu.VMEM((BPW,), jnp.int32),
        pltpu.VMEM((CH, 2 * EMB), jnp.float32),
        pltpu.VMEM((CH, 2 * EMB), jnp.float32),
        pltpu.VMEM((1, EMB), jnp.float32),
        pltpu.VMEM((L,), jnp.float32),
        pltpu.VMEM((BPW,), jnp.float32),
        pltpu.SemaphoreType.DMA,
        pltpu.SemaphoreType.DMA,
        pltpu.SemaphoreType.DMA,
    ],
)
def _sc_kernel(cust_hbm, art_hbm, ctab_hbm, atab_hbm, w_hbm, b_hbm,
               out_hbm, idx_c, idx_a, pid_a, rows_c, rows_a,
               w_v, b_v, out_v, sem_c, sem_c2, sem_a):
    wid = lax.axis_index("s") * NC + lax.axis_index("c")
    base = wid * BPW

    pltpu.sync_copy(cust_hbm.at[pl.ds(base, BPW)], idx_c)
    pltpu.sync_copy(art_hbm.at[pl.ds(base, BPW)], idx_a)
    pltpu.sync_copy(w_hbm, w_v)
    pltpu.sync_copy(b_hbm, b_v.at[pl.ds(0, 1)])

    for i in range(BPW // L):
        sl = pl.ds(i * L, L)
        pid_a[sl] = lax.shift_right_logical(idx_a[sl], 1)

    lane = lax.iota(jnp.int32, L)
    w_vecs = [w_v[0, pl.ds(k * L, L)] for k in range(EMB // L)]
    w_scalars = [w_vecs[d // L][d % L] for d in range(EMB)]
    bias = b_v[...][0]
    one = jnp.full((L,), 1, jnp.int32)

    for h in range(NCH):
        off = h * CH
        cpy_a = pltpu.async_copy(
            atab_hbm.at[pid_a.at[pl.ds(off, CH)]], rows_a, sem_a)
        cpy_a.wait()

        def blk_body(blk, carry):
            row0 = blk * L
            item = row0 + lane
            half_a = lax.bitwise_and(idx_a[pl.ds(off + row0, L)], one) * EMB
            acc = [jnp.full((L,), 0.0, jnp.float32) for _ in range(4)]
            for d in range(EMB):
                dvec = jnp.full((L,), d, jnp.int32)
                cv = plsc.load_gather(rows_c, [item, dvec])
                av = plsc.load_gather(rows_a, [item, half_a + d])
                acc[d % 4] = acc[d % 4] + (cv * av) * w_scalars[d]
            out_v[pl.ds(off + row0, L)] = (
                ((acc[0] + acc[1]) + (acc[2] + acc[3])) + bias)
            return carry

        lax.fori_loop(0, CH // L, blk_body, 0)

    pltpu.sync_copy(out_v, out_hbm.at[pl.ds(base, BPW)])


def kernel(customers, articles, cust_table, art_table, W, b):
    atab2 = art_table.reshape(art_table.shape[0] // 2, 2 * EMB)
    return _sc_kernel(customers.astype(jnp.int32), articles.astype(jnp.int32),
                      cust_table, atab2, W, b)


# speedup vs baseline: 1.0036x; 1.0036x over previous
import functools

import jax
import jax.numpy as jnp
from jax import lax
from jax.experimental import pallas as pl
from jax.experimental.pallas import tpu as pltpu
from jax.experimental.pallas import tpu_sc as plsc

B = 16384
EMB = 64
L = 16
NC = 2
NS = 16
NW = NC * NS
BPW = B // NW
CH = 64
NCH = BPW // CH

_mesh = plsc.VectorSubcoreMesh(core_axis_name="c", subcore_axis_name="s")


@functools.partial(
    pl.kernel,
    mesh=_mesh,
    compiler_params=pltpu.CompilerParams(
        needs_layout_passes=False, use_tc_tiling_on_sc=True),
    out_type=jax.ShapeDtypeStruct((B,), jnp.float32),
    scratch_types=[
        pltpu.VMEM((BPW,), jnp.int32),
        pltpu.VMEM((BPW,), jnp.int32),
        pltpu.VMEM((CH, 2 * EMB), jnp.float32),
        pltpu.VMEM((CH, 2 * EMB), jnp.float32),
        pltpu.VMEM((1, EMB), jnp.float32),
        pltpu.VMEM((L,), jnp.float32),
        pltpu.VMEM((BPW,), jnp.float32),
        pltpu.SemaphoreType.DMA,
        pltpu.SemaphoreType.DMA,
    ],
)
def _sc_kernel(cust_hbm, art_hbm, ctab_hbm, atab_hbm, w_hbm, b_hbm,
               out_hbm, idx_c, idx_a, rows_c, rows_a,
               w_v, b_v, out_v, sem_c, sem_a):
    wid = lax.axis_index("s") * NC + lax.axis_index("c")
    base = wid * BPW

    pltpu.sync_copy(cust_hbm.at[pl.ds(base, BPW)], idx_c)
    pltpu.sync_copy(art_hbm.at[pl.ds(base, BPW)], idx_a)
    pltpu.sync_copy(w_hbm, w_v)
    pltpu.sync_copy(b_hbm, b_v.at[pl.ds(0, 1)])

    lane = lax.iota(jnp.int32, L)
    w_vecs = [w_v[0, pl.ds(k * L, L)] for k in range(EMB // L)]
    w_scalars = [w_vecs[d // L][d % L] for d in range(EMB)]
    bias = b_v[...][0]

    def chunk_body(ch, carry):
        off = ch * CH
        cpys = []
        for g in range(CH // L):
            vc = idx_c[pl.ds(off + g * L, L)]
            va = idx_a[pl.ds(off + g * L, L)]
            for k in range(L):
                j = g * L + k
                cpys.append(pltpu.async_copy(
                    ctab_hbm.at[vc[k]], rows_c.at[j, pl.ds(0, EMB)], sem_c))
                cpys.append(pltpu.async_copy(
                    atab_hbm.at[va[k]], rows_a.at[j, pl.ds(0, EMB)], sem_a))
        for cp in cpys:
            cp.wait()
        for g in range(CH // L):
            item = jnp.full((L,), g * L, jnp.int32) + lane
            acc = [jnp.full((L,), 0.0, jnp.float32) for _ in range(4)]
            for d in range(EMB):
                dvec = jnp.full((L,), d, jnp.int32)
                cv = plsc.load_gather(rows_c, [item, dvec])
                av = plsc.load_gather(rows_a, [item, dvec])
                acc[d % 4] = acc[d % 4] + (cv * av) * w_scalars[d]
            out_v[pl.ds(off + g * L, L)] = (
                ((acc[0] + acc[1]) + (acc[2] + acc[3])) + bias)
        return carry

    lax.fori_loop(0, NCH, chunk_body, 0)

    pltpu.sync_copy(out_v, out_hbm.at[pl.ds(base, BPW)])


def kernel(customers, articles, cust_table, art_table, W, b):
    return _sc_kernel(customers.astype(jnp.int32), articles.astype(jnp.int32),
                      cust_table, art_table, W, b)


# speedup vs baseline: 1.0596x; 1.0558x over previous
import functools

import jax
import jax.numpy as jnp
from jax import lax
from jax.experimental import pallas as pl
from jax.experimental.pallas import tpu as pltpu
from jax.experimental.pallas import tpu_sc as plsc

B = 16384
EMB = 64
L = 16
NC = 2
NS = 16
NW = NC * NS
BPW = B // NW
CH = 64
NCH = BPW // CH

_mesh = plsc.VectorSubcoreMesh(core_axis_name="c", subcore_axis_name="s")


@functools.partial(
    pl.kernel,
    mesh=_mesh,
    compiler_params=pltpu.CompilerParams(
        needs_layout_passes=False, use_tc_tiling_on_sc=True),
    out_type=jax.ShapeDtypeStruct((B,), jnp.float32),
    scratch_types=[
        pltpu.VMEM((BPW,), jnp.int32),
        pltpu.VMEM((BPW,), jnp.int32),
        pltpu.VMEM((CH, 2 * EMB), jnp.float32),
        pltpu.VMEM((CH, 2 * EMB), jnp.float32),
        pltpu.VMEM((1, EMB), jnp.float32),
        pltpu.VMEM((L,), jnp.float32),
        pltpu.VMEM((BPW,), jnp.float32),
        pltpu.SemaphoreType.DMA,
        pltpu.SemaphoreType.DMA,
    ],
)
def _sc_kernel(cust_hbm, art_hbm, ctab_hbm, atab_hbm, w_hbm, b_hbm,
               out_hbm, idx_c, idx_a, rows_c, rows_a,
               w_v, b_v, out_v, sem_c, sem_a):
    wid = lax.axis_index("s") * NC + lax.axis_index("c")
    base = wid * BPW

    pltpu.sync_copy(cust_hbm.at[pl.ds(base, BPW)], idx_c)
    pltpu.sync_copy(art_hbm.at[pl.ds(base, BPW)], idx_a)
    pltpu.sync_copy(w_hbm, w_v)
    pltpu.sync_copy(b_hbm, b_v.at[pl.ds(0, 1)])

    lane = lax.iota(jnp.int32, L)
    w_vecs = [w_v[0, pl.ds(k * L, L)] for k in range(EMB // L)]
    w_scalars = [w_vecs[d // L][d % L] for d in range(EMB)]
    bias = b_v[...][0]

    def chunk_body(ch, carry):
        off = ch * CH
        cpys = []
        for g in range(CH // L):
            vc = idx_c[pl.ds(off + g * L, L)]
            va = idx_a[pl.ds(off + g * L, L)]
            for k in range(L):
                j = g * L + k
                cpys.append(pltpu.async_copy(
                    ctab_hbm.at[vc[k]], rows_c.at[j, pl.ds(0, EMB)], sem_c))
                cpys.append(pltpu.async_copy(
                    atab_hbm.at[va[k]], rows_a.at[j, pl.ds(0, EMB)], sem_a))
        for cp in cpys:
            cp.wait()
        for g in range(CH // L):
            res = jnp.full((L,), 0.0, jnp.float32)
            for k in range(L):
                j = g * L + k
                parts = []
                for q in range(EMB // L):
                    cv = rows_c[j, pl.ds(q * L, L)]
                    av = rows_a[j, pl.ds(q * L, L)]
                    parts.append((cv * av) * w_vecs[q])
                s = (parts[0] + parts[1]) + (parts[2] + parts[3])
                res = jnp.where(lane == k, jnp.sum(s), res)
            out_v[pl.ds(off + g * L, L)] = res + bias
        return carry

    lax.fori_loop(0, NCH, chunk_body, 0)

    pltpu.sync_copy(out_v, out_hbm.at[pl.ds(base, BPW)])


def kernel(customers, articles, cust_table, art_table, W, b):
    return _sc_kernel(customers.astype(jnp.int32), articles.astype(jnp.int32),
                      cust_table, art_table, W, b)


# speedup vs baseline: 1.0623x; 1.0025x over previous
import functools

import jax
import jax.numpy as jnp
from jax import lax
from jax.experimental import pallas as pl
from jax.experimental.pallas import tpu as pltpu
from jax.experimental.pallas import tpu_sc as plsc

B = 16384
EMB = 64
L = 16
NC = 2
NS = 16
NW = NC * NS
BPW = B // NW
CH = 64
NCH = BPW // CH

_mesh = plsc.VectorSubcoreMesh(core_axis_name="c", subcore_axis_name="s")


@functools.partial(
    pl.kernel,
    mesh=_mesh,
    compiler_params=pltpu.CompilerParams(
        needs_layout_passes=False, use_tc_tiling_on_sc=True),
    out_type=jax.ShapeDtypeStruct((B,), jnp.float32),
    scratch_types=[
        pltpu.VMEM((BPW,), jnp.int32),
        pltpu.VMEM((BPW,), jnp.int32),
        pltpu.VMEM((CH, 2 * EMB), jnp.float32),
        pltpu.VMEM((CH, 2 * EMB), jnp.float32),
        pltpu.VMEM((1, EMB), jnp.float32),
        pltpu.VMEM((L,), jnp.float32),
        pltpu.VMEM((BPW,), jnp.float32),
        pltpu.SemaphoreType.DMA,
        pltpu.SemaphoreType.DMA,
    ],
)
def _sc_kernel(cust_hbm, art_hbm, ctab_hbm, atab_hbm, w_hbm, b_hbm,
               out_hbm, idx_c, idx_a, rows_c, rows_a,
               w_v, b_v, out_v, sem_c, sem_a):
    wid = lax.axis_index("s") * NC + lax.axis_index("c")
    base = wid * BPW

    pltpu.sync_copy(cust_hbm.at[pl.ds(base, BPW)], idx_c)
    pltpu.sync_copy(art_hbm.at[pl.ds(base, BPW)], idx_a)
    pltpu.sync_copy(w_hbm, w_v)
    pltpu.sync_copy(b_hbm, b_v.at[pl.ds(0, 1)])

    lane = lax.iota(jnp.int32, L)
    w_vecs = [w_v[0, pl.ds(k * L, L)] for k in range(EMB // L)]
    bias = b_v[...][0]

    def chunk_body(ch, carry):
        off = ch * CH
        cpys = []
        for g in range(CH // L):
            vc = idx_c[pl.ds(off + g * L, L)]
            va = idx_a[pl.ds(off + g * L, L)]
            for k in range(L):
                j = g * L + k
                cpys.append(pltpu.async_copy(
                    ctab_hbm.at[vc[k]], rows_c.at[j, pl.ds(0, EMB)], sem_c))
                cpys.append(pltpu.async_copy(
                    atab_hbm.at[va[k]], rows_a.at[j, pl.ds(0, EMB)], sem_a))
        for cp in cpys:
            cp.wait()
        for g in range(CH // L):
            res = jnp.full((L,), 0.0, jnp.float32)
            for k in range(L):
                j = g * L + k
                parts = []
                for q in range(EMB // L):
                    cv = rows_c[j, pl.ds(q * L, L)]
                    av = rows_a[j, pl.ds(q * L, L)]
                    parts.append((cv * av) * w_vecs[q])
                s = (parts[0] + parts[1]) + (parts[2] + parts[3])
                res = jnp.where(lane == k, jnp.sum(s), res)
            out_v[pl.ds(off + g * L, L)] = res + bias
        return carry

    lax.fori_loop(0, NCH, chunk_body, 0)

    pltpu.sync_copy(out_v, out_hbm.at[pl.ds(base, BPW)])


def kernel(customers, articles, cust_table, art_table, W, b):
    return _sc_kernel(customers.astype(jnp.int32), articles.astype(jnp.int32),
                      cust_table, art_table, W, b)
